# Initial kernel scaffold; baseline (speedup 1.0000x reference)
import functools

import jax
import jax.numpy as jnp
import numpy as np
from jax import lax
from jax.experimental import pallas as pl
from jax.experimental.pallas import tpu as pltpu
from jax.experimental.pallas import tpu_sc as plsc

_AUG_RATIO = 0.2
_N_EDGES = 320000
_N_NODES = 10000
_PERM = int(_N_EDGES * _AUG_RATIO)
_KEEP = _N_EDGES - _PERM

_NW = 32
_GCH = (2 * _KEEP) // _NW
_ACH = (2 * _PERM) // _NW


@functools.cache
def _consts():
    key = jax.random.key(42)
    ka, kb = jax.random.split(key)
    add = jax.random.randint(ka, (2, _PERM), 0, _N_NODES, dtype=jnp.int32)
    keep = jax.random.permutation(kb, _N_EDGES)[:_KEEP]
    keep = np.asarray(keep).astype(np.int32)
    idxg = np.concatenate([keep, keep + _N_EDGES]).astype(np.int32)
    addf = np.asarray(add).astype(np.int32).reshape(-1)
    return idxg, addf


def _augment_edges(flat_edges, idxg, addf):
    mesh = plsc.VectorSubcoreMesh(core_axis_name="c", subcore_axis_name="s")

    @functools.partial(
        pl.kernel,
        out_type=jax.ShapeDtypeStruct((2 * _N_EDGES,), jnp.int32),
        mesh=mesh,
        scratch_types=[
            pltpu.VMEM((_GCH,), jnp.int32),
            pltpu.VMEM((_GCH,), jnp.int32),
            pltpu.VMEM((_ACH,), jnp.int32),
            pltpu.SemaphoreType.DMA,
        ],
    )
    def body(edges_hbm, idxg_hbm, add_hbm, out_hbm, idx_v, gat_v, add_v, sem):
        w = lax.axis_index("s") * 2 + lax.axis_index("c")
        gbase = w * _GCH
        pltpu.sync_copy(idxg_hbm.at[pl.ds(gbase, _GCH)], idx_v)
        pltpu.async_copy(edges_hbm.at[idx_v], gat_v, sem).wait()
        gshift = jnp.where(w < _NW // 2, 0, _PERM)
        obase = pl.multiple_of(gbase + gshift, 8)
        pltpu.sync_copy(gat_v, out_hbm.at[pl.ds(obase, _GCH)])
        abase = w * _ACH
        pltpu.sync_copy(add_hbm.at[pl.ds(abase, _ACH)], add_v)
        ashift = jnp.where(w < _NW // 2, 0, _KEEP)
        aoff = pl.multiple_of(_KEEP + abase + ashift, 8)
        pltpu.sync_copy(add_v, out_hbm.at[pl.ds(aoff, _ACH)])

    return body(flat_edges, idxg, addf)


def kernel(x, edge_index, root_index):
    idxg, addf = _consts()
    flat = edge_index.reshape(2 * _N_EDGES)
    out_flat = _augment_edges(flat, jnp.asarray(idxg), jnp.asarray(addf))
    return x, out_flat.reshape(2, _N_EDGES)


# speedup vs baseline: 15.4728x; 15.4728x over previous
import functools

import jax
import jax.numpy as jnp
import numpy as np
from jax import lax
from jax.experimental import pallas as pl
from jax.experimental.pallas import tpu as pltpu
from jax.experimental.pallas import tpu_sc as plsc

_AUG_RATIO = 0.2
_N_EDGES = 320000
_N_NODES = 10000
_PERM = int(_N_EDGES * _AUG_RATIO)
_KEEP = _N_EDGES - _PERM

_NW = 32
_GCH = (2 * _KEEP) // _NW
_ACH = (2 * _PERM) // _NW


_U32 = np.uint32


def _threefry2x32(k1, k2, x1, x2):
    def rotl(x, d):
        return ((x << _U32(d)) | (x >> _U32(32 - d))).astype(np.uint32)

    rot = [(13, 15, 26, 6), (17, 29, 16, 24)]
    ks = [k1, k2, (k1 ^ k2 ^ _U32(0x1BD11BDA)).astype(np.uint32)]
    x = [(x1 + ks[0]).astype(np.uint32), (x2 + ks[1]).astype(np.uint32)]
    for i in range(5):
        for r in rot[i % 2]:
            x[0] = (x[0] + x[1]).astype(np.uint32)
            x[1] = (x[0] ^ rotl(x[1], r)).astype(np.uint32)
        x[0] = (x[0] + ks[(i + 1) % 3]).astype(np.uint32)
        x[1] = (x[1] + ks[(i + 2) % 3] + _U32(i + 1)).astype(np.uint32)
    return x[0], x[1]


def _iota_2x32(n):
    c = np.arange(n, dtype=np.uint64)
    return (c >> np.uint64(32)).astype(np.uint32), c.astype(np.uint32)


def _random_bits32(key, n):
    hi, lo = _iota_2x32(n)
    b1, b2 = _threefry2x32(key[0], key[1], hi, lo)
    return (b1 ^ b2).astype(np.uint32)


def _split2(key):
    hi, lo = _iota_2x32(2)
    b1, b2 = _threefry2x32(key[0], key[1], hi, lo)
    return (b1[0], b2[0]), (b1[1], b2[1])


def _np_randint(key, n, span):
    k1, k2 = _split2(key)
    higher = _random_bits32(k1, n)
    lower = _random_bits32(k2, n)
    span = _U32(span)
    mult = _U32((int(_U32(2 ** 16) % span) ** 2) % int(span))
    off = ((higher % span) * mult + lower % span) % span
    return off.astype(np.int32)


def _np_permutation(key, n):
    x = np.arange(n, dtype=np.int32)
    num_rounds = int(np.ceil(3 * np.log(max(1, n)) / np.log(np.iinfo(np.uint32).max)))
    for _ in range(num_rounds):
        key, sub = _split2(key)
        sort_keys = _random_bits32(sub, n)
        x = x[np.argsort(sort_keys, kind="stable")]
    return x


@functools.cache
def _consts():
    ka, kb = _split2((_U32(0), _U32(42)))
    add = _np_randint(ka, 2 * _PERM, _N_NODES).reshape(2, _PERM)
    keep = _np_permutation(kb, _N_EDGES)[:_KEEP].astype(np.int32)
    idxg = np.concatenate([keep, keep + _N_EDGES]).astype(np.int32)
    addf = add.astype(np.int32).reshape(-1)
    return idxg, addf


def _augment_edges(flat_edges, idxg, addf):
    mesh = plsc.VectorSubcoreMesh(core_axis_name="c", subcore_axis_name="s")

    @functools.partial(
        pl.kernel,
        out_type=jax.ShapeDtypeStruct((2 * _N_EDGES,), jnp.int32),
        mesh=mesh,
        scratch_types=[
            pltpu.VMEM((_GCH,), jnp.int32),
            pltpu.VMEM((_GCH,), jnp.int32),
            pltpu.VMEM((_ACH,), jnp.int32),
            pltpu.SemaphoreType.DMA,
        ],
    )
    def body(edges_hbm, idxg_hbm, add_hbm, out_hbm, idx_v, gat_v, add_v, sem):
        w = lax.axis_index("s") * 2 + lax.axis_index("c")
        gbase = w * _GCH
        pltpu.sync_copy(idxg_hbm.at[pl.ds(gbase, _GCH)], idx_v)
        pltpu.async_copy(edges_hbm.at[idx_v], gat_v, sem).wait()
        gshift = jnp.where(w < _NW // 2, 0, _PERM)
        obase = pl.multiple_of(gbase + gshift, 8)
        pltpu.sync_copy(gat_v, out_hbm.at[pl.ds(obase, _GCH)])
        abase = w * _ACH
        pltpu.sync_copy(add_hbm.at[pl.ds(abase, _ACH)], add_v)
        ashift = jnp.where(w < _NW // 2, 0, _KEEP)
        aoff = pl.multiple_of(_KEEP + abase + ashift, 8)
        pltpu.sync_copy(add_v, out_hbm.at[pl.ds(aoff, _ACH)])

    return body(flat_edges, idxg, addf)


def kernel(x, edge_index, root_index):
    idxg, addf = _consts()
    flat = edge_index.reshape(2 * _N_EDGES)
    out_flat = _augment_edges(flat, jnp.asarray(idxg), jnp.asarray(addf))
    return x, out_flat.reshape(2, _N_EDGES)


# speedup vs baseline: 17.0334x; 1.1009x over previous
import functools

import jax
import jax.numpy as jnp
import numpy as np
from jax import lax
from jax.experimental import pallas as pl
from jax.experimental.pallas import tpu as pltpu
from jax.experimental.pallas import tpu_sc as plsc

_AUG_RATIO = 0.2
_N_EDGES = 320000
_N_NODES = 10000
_PERM = int(_N_EDGES * _AUG_RATIO)
_KEEP = _N_EDGES - _PERM

_NW = 32
_GCH = (2 * _KEEP) // _NW
_ACH = (2 * _PERM) // _NW


_U32 = np.uint32


def _threefry2x32(k1, k2, x1, x2):
    def rotl(x, d):
        return ((x << _U32(d)) | (x >> _U32(32 - d))).astype(np.uint32)

    rot = [(13, 15, 26, 6), (17, 29, 16, 24)]
    ks = [k1, k2, (k1 ^ k2 ^ _U32(0x1BD11BDA)).astype(np.uint32)]
    x = [(x1 + ks[0]).astype(np.uint32), (x2 + ks[1]).astype(np.uint32)]
    for i in range(5):
        for r in rot[i % 2]:
            x[0] = (x[0] + x[1]).astype(np.uint32)
            x[1] = (x[0] ^ rotl(x[1], r)).astype(np.uint32)
        x[0] = (x[0] + ks[(i + 1) % 3]).astype(np.uint32)
        x[1] = (x[1] + ks[(i + 2) % 3] + _U32(i + 1)).astype(np.uint32)
    return x[0], x[1]


def _iota_2x32(n):
    c = np.arange(n, dtype=np.uint64)
    return (c >> np.uint64(32)).astype(np.uint32), c.astype(np.uint32)


def _random_bits32(key, n):
    hi, lo = _iota_2x32(n)
    b1, b2 = _threefry2x32(key[0], key[1], hi, lo)
    return (b1 ^ b2).astype(np.uint32)


def _split2(key):
    hi, lo = _iota_2x32(2)
    b1, b2 = _threefry2x32(key[0], key[1], hi, lo)
    return (b1[0], b2[0]), (b1[1], b2[1])


def _np_randint(key, n, span):
    k1, k2 = _split2(key)
    higher = _random_bits32(k1, n)
    lower = _random_bits32(k2, n)
    span = _U32(span)
    mult = _U32((int(_U32(2 ** 16) % span) ** 2) % int(span))
    off = ((higher % span) * mult + lower % span) % span
    return off.astype(np.int32)


def _np_permutation(key, n):
    x = np.arange(n, dtype=np.int32)
    num_rounds = int(np.ceil(3 * np.log(max(1, n)) / np.log(np.iinfo(np.uint32).max)))
    for _ in range(num_rounds):
        key, sub = _split2(key)
        sort_keys = _random_bits32(sub, n)
        x = x[np.argsort(sort_keys, kind="stable")]
    return x


@functools.cache
def _consts():
    ka, kb = _split2((_U32(0), _U32(42)))
    add = _np_randint(ka, 2 * _PERM, _N_NODES).reshape(2, _PERM)
    keep = _np_permutation(kb, _N_EDGES)[:_KEEP].astype(np.int64)

    def tiled_addr(r, e):
        return (e >> 7) * 256 + r * 128 + (e & 127)

    j = np.arange(2 * _KEEP, dtype=np.int64)
    r = (j >> 7) & 1
    o = (j >> 8) * 128 + (j & 127)
    idxg = (r * _N_EDGES + keep[o]).astype(np.int32)
    ja = np.arange(2 * _KEEP, 2 * _N_EDGES, dtype=np.int64)
    ra = (ja >> 7) & 1
    oa = (ja >> 8) * 128 + (ja & 127)
    addf = add[ra, oa - _KEEP].astype(np.int32)
    return idxg, addf


def _augment_edges(flat_edges, idxg, addf):
    mesh = plsc.VectorSubcoreMesh(core_axis_name="c", subcore_axis_name="s")

    @functools.partial(
        pl.kernel,
        out_type=jax.ShapeDtypeStruct((2 * _N_EDGES,), jnp.int32),
        mesh=mesh,
        scratch_types=[
            pltpu.VMEM((_GCH,), jnp.int32),
            pltpu.VMEM((_GCH,), jnp.int32),
            pltpu.VMEM((_ACH,), jnp.int32),
            pltpu.SemaphoreType.DMA,
        ],
    )
    def body(edges_hbm, idxg_hbm, add_hbm, out_hbm, idx_v, gat_v, add_v, sem):
        w = lax.axis_index("s") * 2 + lax.axis_index("c")
        gbase = w * _GCH
        pltpu.sync_copy(idxg_hbm.at[pl.ds(gbase, _GCH)], idx_v)
        pltpu.async_copy(edges_hbm.at[idx_v], gat_v, sem).wait()
        pltpu.sync_copy(gat_v, out_hbm.at[pl.ds(gbase, _GCH)])
        abase = w * _ACH
        pltpu.sync_copy(add_hbm.at[pl.ds(abase, _ACH)], add_v)
        pltpu.sync_copy(add_v, out_hbm.at[pl.ds(2 * _KEEP + abase, _ACH)])

    return body(flat_edges, idxg, addf)


def kernel(x, edge_index, root_index):
    idxg, addf = _consts()
    flat = edge_index.reshape(2 * _N_EDGES)
    out_flat = _augment_edges(flat, jnp.asarray(idxg), jnp.asarray(addf))
    out = out_flat.reshape(_N_EDGES // 128, 2, 128).transpose(1, 0, 2)
    return x, out.reshape(2, _N_EDGES)
